# baseline (speedup 1.0000x reference)
import functools

import jax
import jax.numpy as jnp
import numpy as np
from jax import lax
from jax.experimental import pallas as pl
from jax.experimental.pallas import tpu as pltpu
from jax.experimental.pallas import tpu_sc as plsc

B, D, N = 8, 256, 40962
P = (N + 6) // 4
K = 7
L = 16
NUM_CORES = 2
NUM_SUBCORES = 16
NW = NUM_CORES * NUM_SUBCORES
G = 4
NCHUNK = (P + G - 1) // G
T_ITERS = (NCHUNK + NW - 1) // NW
CPAD = T_ITERS * NW
IDXW = 64


def _pool_body(xt_hbm, idxt_hbm, out_hbm, idx_v, planes, outb, gsem, osem):
    wid = lax.axis_index("s") * NUM_CORES + lax.axis_index("c")
    inv_k = jnp.float32(1.0 / K)

    pltpu.sync_copy(idxt_hbm.at[wid], idx_v)

    def issue_gathers(t, st):
        off = pl.multiple_of(t * IDXW, 8)
        pltpu.async_copy(
            xt_hbm.at[idx_v.at[pl.ds(off, K * G)]], planes.at[st], gsem)

    issue_gathers(0, 0)

    def halfstep(t, st):
        c = wid + NW * t

        @pl.when(c < NCHUNK)
        def _():
            @pl.when(wid + NW * (t + 1) < NCHUNK)
            def _():
                issue_gathers(t + 1, 1 - st)

            pltpu.make_async_copy(
                xt_hbm.at[pl.ds(0, K * G)], planes.at[0], gsem).wait()

            @pl.when(t > 0)
            def _():
                pltpu.make_async_copy(
                    outb, out_hbm.at[pl.ds(0, G)], osem).wait()

            def fuse(i, cc):
                g = i // B
                b_ = i % B
                for t16 in range(D // L):
                    sl = pl.ds(t16 * L, L)
                    acc = planes[st, g, b_, sl]
                    for j in range(1, K):
                        acc = acc + planes[st, j * G + g, b_, sl]
                    outb[g, b_, sl] = acc * inv_k
                return cc

            lax.fori_loop(0, G * B, fuse, 0)

            s = jnp.minimum(c * G, P - G)
            pltpu.async_copy(outb, out_hbm.at[pl.ds(s, G)], osem)

    def loop_body(u, carry):
        halfstep(2 * u, 0)
        halfstep(2 * u + 1, 1)
        return carry

    lax.fori_loop(0, T_ITERS // 2, loop_body, 0)
    halfstep(T_ITERS - 1, 0)
    pltpu.make_async_copy(outb, out_hbm.at[pl.ds(0, G)], osem).wait()


def _pos_map() -> np.ndarray:
    c = np.arange(CPAD)
    s = np.minimum(c * G, P - G)
    pos = ((s[:, None, None] + np.arange(G)[None, None, :]) * K
           + np.arange(K)[None, :, None])
    pos = pos.reshape(CPAD, K * G)
    pos = np.pad(pos, ((0, 0), (0, IDXW - K * G)))
    return (pos.reshape(T_ITERS, NW, IDXW)
            .transpose(1, 0, 2).reshape(NW, T_ITERS * IDXW).astype(np.int32))


_POS_MAP = _pos_map()


@functools.partial(jax.jit, static_argnames=())
def kernel(x, neigh_orders):
    idx = neigh_orders[:P, :].astype(jnp.int32)
    idx_tiles = jnp.take(idx.reshape(-1), jnp.asarray(_POS_MAP))

    xt = x.transpose(2, 0, 1)

    pool = pl.kernel(
        _pool_body,
        out_type=jax.ShapeDtypeStruct((P, B, D), jnp.float32),
        mesh=plsc.VectorSubcoreMesh(
            core_axis_name="c", subcore_axis_name="s",
            num_cores=NUM_CORES, num_subcores=NUM_SUBCORES),
        scratch_types=[
            pltpu.VMEM((T_ITERS * IDXW,), jnp.int32),
            pltpu.VMEM((2, K * G, B, D), jnp.float32),
            pltpu.VMEM((G, B, D), jnp.float32),
            pltpu.SemaphoreType.DMA,
            pltpu.SemaphoreType.DMA,
        ],
        compiler_params=pltpu.CompilerParams(needs_layout_passes=False),
    )
    out_t = pool(xt, idx_tiles)
    return out_t.transpose(1, 2, 0)


# speedup vs baseline: 4.4110x; 4.4110x over previous
import functools

import jax
import jax.numpy as jnp
import numpy as np
from jax import lax
from jax.experimental import pallas as pl
from jax.experimental.pallas import tpu as pltpu
from jax.experimental.pallas import tpu_sc as plsc

B, D, N = 8, 256, 40962
P = (N + 6) // 4
K = 7
L = 16
NUM_CORES = 2
NUM_SUBCORES = 16
NW = NUM_CORES * NUM_SUBCORES
G = 4
NCHUNK = (P + G - 1) // G
T_ITERS = (NCHUNK + NW - 1) // NW
CPAD = T_ITERS * NW
IDXW = 64


def _pool_body(xt_hbm, idxt_hbm, out_hbm, idx_v, planes, outb, gsem, osem):
    wid = lax.axis_index("s") * NUM_CORES + lax.axis_index("c")
    inv_k = jnp.float32(1.0 / K)

    pltpu.sync_copy(idxt_hbm.at[wid], idx_v)

    def issue_gathers(t, st):
        off = pl.multiple_of(t * IDXW, 8)
        pltpu.async_copy(
            xt_hbm.at[idx_v.at[pl.ds(off, K * G)]], planes.at[st], gsem)

    issue_gathers(0, 0)

    def halfstep(t, st):
        c = wid + NW * t

        @pl.when(c < NCHUNK)
        def _():
            @pl.when(wid + NW * (t + 1) < NCHUNK)
            def _():
                issue_gathers(t + 1, 1 - st)

            pltpu.make_async_copy(
                xt_hbm.at[pl.ds(0, K * G)], planes.at[0], gsem).wait()

            @pl.when(t > 0)
            def _():
                pltpu.make_async_copy(
                    outb, out_hbm.at[pl.ds(0, G)], osem).wait()

            def fuse(i, cc):
                g = i // B
                b_ = i % B
                for t16 in range(D // L):
                    sl = pl.ds(t16 * L, L)
                    acc = planes[st, g, b_, sl]
                    for j in range(1, K):
                        acc = acc + planes[st, j * G + g, b_, sl]
                    outb[g, b_, sl] = acc * inv_k
                return cc

            lax.fori_loop(0, G * B, fuse, 0)

            s = jnp.minimum(c * G, P - G)
            pltpu.async_copy(outb, out_hbm.at[pl.ds(s, G)], osem)

    def loop_body(u, carry):
        halfstep(2 * u, 0)
        halfstep(2 * u + 1, 1)
        return carry

    lax.fori_loop(0, T_ITERS // 2, loop_body, 0)
    halfstep(T_ITERS - 1, 0)
    pltpu.make_async_copy(outb, out_hbm.at[pl.ds(0, G)], osem).wait()


@functools.partial(jax.jit, static_argnames=())
def kernel(x, neigh_orders):
    idx = neigh_orders[:P, :].astype(jnp.int32)
    starts = jnp.minimum(jnp.arange(CPAD) * G, P - G)
    pos = starts[:, None] + jnp.arange(G)[None, :]
    blk = idx[pos].transpose(0, 2, 1)
    blk = jnp.pad(blk.reshape(CPAD, K * G), ((0, 0), (0, IDXW - K * G)))
    idx_tiles = (blk.reshape(T_ITERS, NW, IDXW)
                 .transpose(1, 0, 2).reshape(NW, T_ITERS * IDXW))

    xt = x.transpose(2, 0, 1)

    pool = pl.kernel(
        _pool_body,
        out_type=jax.ShapeDtypeStruct((P, B, D), jnp.float32),
        mesh=plsc.VectorSubcoreMesh(
            core_axis_name="c", subcore_axis_name="s",
            num_cores=NUM_CORES, num_subcores=NUM_SUBCORES),
        scratch_types=[
            pltpu.VMEM((T_ITERS * IDXW,), jnp.int32),
            pltpu.VMEM((2, K * G, B, D), jnp.float32),
            pltpu.VMEM((G, B, D), jnp.float32),
            pltpu.SemaphoreType.DMA,
            pltpu.SemaphoreType.DMA,
        ],
        compiler_params=pltpu.CompilerParams(needs_layout_passes=False),
    )
    out_t = pool(xt, idx_tiles)
    return out_t.transpose(1, 2, 0)
